# baseline (speedup 1.0000x reference)
import math

import numpy as np

import jax
import jax.numpy as jnp
from jax import lax
from jax.experimental import pallas as pl
from jax.experimental.pallas import tpu as pltpu
from jax.experimental.pallas import tpu_sc as plsc

N = 10000
E = 320000
DIN = 128
DH = 128
DO = 64
DT = 64

LANE = 128
NW = 32
RW = 80
RPAD = NW * RW
EPAD = RPAD * LANE
GPAD = 112
NACC = N + GPAD
TROWS = NACC // 16
WCH = (128, 128, 128, 128, 120)

COEFF = math.exp(-1.0)
SELU_ALPHA = 1.6732632423543772848170429916717
SELU_SCALE = 1.0507009873554804934193349852946

_mesh = lambda: plsc.VectorSubcoreMesh(core_axis_name="c", subcore_axis_name="s")
_SC_PARAMS = pltpu.CompilerParams(use_tc_tiling_on_sc=False)


DW = 16


def _deg_body(dstr, onesc, zcol, out_hbm, didx, obuf, zbuf, acc):
    c = lax.axis_index("c")
    s = lax.axis_index("s")
    wid = s * 2 + c
    pltpu.sync_copy(dstr.at[pl.ds(wid * RW, RW)], didx)
    pltpu.sync_copy(onesc, obuf)
    pltpu.sync_copy(zcol, zbuf)
    rbase = s * TROWS
    pltpu.sync_copy(zbuf, acc.at[pl.ds(rbase, TROWS)])
    plsc.subcore_barrier()

    def step(j, carry):
        pltpu.sync_copy(obuf, acc.at[didx.at[j]], add=True)
        return carry

    lax.fori_loop(0, RW, step, 0)
    plsc.subcore_barrier()
    pltpu.sync_copy(acc.at[pl.ds(rbase, TROWS)], zbuf)
    pltpu.sync_copy(zbuf, out_hbm.at[pl.ds(rbase, TROWS), pl.ds(DW * c, DW)])


def _make_deg():
    return pl.kernel(
        _deg_body,
        out_type=jax.ShapeDtypeStruct((NACC, 2 * DW), jnp.float32),
        mesh=_mesh(),
        compiler_params=_SC_PARAMS,
        scratch_types=[
            pltpu.VMEM((RW, LANE), jnp.int32),
            pltpu.VMEM((LANE, DW), jnp.float32),
            pltpu.VMEM((TROWS, DW), jnp.float32),
            pltpu.VMEM_SHARED((NACC, DW), jnp.float32),
        ],
    )


NROWS = N // 16
LCH = (128, 128, 128, 128, 113)


def _make_prop(ntab):
    def body(*refs):
        tabs = refs[:ntab]
        (srcr, dstr, zrow, out_hbm, sidx, didx, buf0, buf1,
         acc, ytab, sg0, sg1, ss0, ss1) = refs[ntab:]
        bufs = (buf0, buf1)
        sg = (sg0, sg1)
        ss = (ss0, ss1)
        c = lax.axis_index("c")
        s = lax.axis_index("s")
        wid = s * 2 + c
        base = wid * RW
        pltpu.sync_copy(srcr.at[pl.ds(base, RW)], sidx)
        pltpu.sync_copy(dstr.at[pl.ds(base, RW)], didx)
        rbase = s * TROWS
        pltpu.sync_copy(zrow, buf0)

        def zero_own_slice():
            off = 0
            for csz in WCH:
                pltpu.sync_copy(buf0.at[pl.ds(0, csz)],
                                acc.at[pl.ds(rbase + off, csz)])
                off += csz

        def load_table():
            off = 0
            for csz in LCH:
                pltpu.sync_copy(tab.at[pl.ds(s * NROWS + off, csz)],
                                buf1.at[pl.ds(0, csz)])
                pltpu.sync_copy(buf1.at[pl.ds(0, csz)],
                                ytab.at[pl.ds(s * NROWS + off, csz)])
                off += csz

        def gather(jj, b):
            return pltpu.async_copy(ytab.at[sidx.at[jj]], bufs[b], sg[b])

        def scatter(jj, b):
            return pltpu.async_copy(bufs[b], acc.at[didx.at[jj]], ss[b],
                                    add=True)

        def wait_gather(jj, b):
            pltpu.make_async_copy(ytab.at[sidx.at[jj]], bufs[b], sg[b]).wait()

        def wait_scatter(jj, b):
            pltpu.make_async_copy(bufs[b], acc.at[didx.at[jj]], ss[b]).wait()

        zero_own_slice()
        for t in range(ntab):
            tab = tabs[t]
            load_table()
            plsc.subcore_barrier()

            gather(0, 0)
            gather(1, 1)
            wait_gather(0, 0)
            scatter(0, 0)
            wait_gather(1, 1)
            scatter(1, 1)

            def step(i, carry):
                j0 = 2 * i
                for b in range(2):
                    j = j0 + b
                    wait_scatter(j - 2, b)
                    gather(j, b)
                    wait_gather(j, b)
                    scatter(j, b)
                return carry

            lax.fori_loop(1, RW // 2, step, 0)
            wait_scatter(RW - 2, 0)
            wait_scatter(RW - 1, 1)
            plsc.subcore_barrier()
            off = 0
            for csz in WCH:
                pltpu.sync_copy(acc.at[pl.ds(rbase + off, csz)],
                                buf1.at[pl.ds(0, csz)])
                pltpu.sync_copy(
                    buf1.at[pl.ds(0, csz)],
                    out_hbm.at[t, pl.ds(rbase + off, csz), pl.ds(DT * c, DT)])
                off += csz
            if t + 1 < ntab:
                pltpu.sync_copy(zrow, buf0)
                zero_own_slice()

    return pl.kernel(
        body,
        out_type=jax.ShapeDtypeStruct((ntab, NACC, 2 * DT), jnp.float32),
        mesh=_mesh(),
        compiler_params=_SC_PARAMS,
        scratch_types=[
            pltpu.VMEM((RW, LANE), jnp.int32),
            pltpu.VMEM((RW, LANE), jnp.int32),
            pltpu.VMEM((LANE, DT), jnp.float32),
            pltpu.VMEM((LANE, DT), jnp.float32),
            pltpu.VMEM_SHARED((NACC, DT), jnp.float32),
            pltpu.VMEM_SHARED((N, DT), jnp.float32),
            pltpu.SemaphoreType.DMA,
            pltpu.SemaphoreType.DMA,
            pltpu.SemaphoreType.DMA,
            pltpu.SemaphoreType.DMA,
        ],
    )


RB = 2000
GRID = N // RB


def _k1_body(degp, x, y0lo, y0hi, d1, d2):
    deg = degp[:, :1] + degp[:, DW:DW + 1]
    dinv1 = jnp.where(deg > 0, lax.rsqrt(jnp.maximum(deg, 1e-12)), 0.0)
    d1[...] = dinv1
    d2[...] = lax.rsqrt(deg + 1.0)
    y0 = x[...] * dinv1
    y0lo[...] = y0[:, :DT]
    y0hi[...] = y0[:, DT:]


def _k2_body(z1p, d1, y1lo, y1hi):
    dd = d1[...] * d1[...]
    y1lo[...] = (z1p[0, :, :DT] + z1p[0, :, DT:]) * dd
    y1hi[...] = (z1p[1, :, :DT] + z1p[1, :, DT:]) * dd


def _k3_body(z2p, d1, d2, W1, b1, W2, g):
    s = d1[...] * COEFF
    x2 = jnp.concatenate(
        [z2p[0, :, :DT] + z2p[0, :, DT:],
         z2p[1, :, :DT] + z2p[1, :, DT:]], axis=1) * s
    h = jnp.dot(x2, W1[...], preferred_element_type=jnp.float32) + b1[...]
    h = SELU_SCALE * jnp.where(h > 0, h, SELU_ALPHA * (jnp.exp(h) - 1.0))
    g[...] = d2[...] * jnp.dot(h, W2[...], preferred_element_type=jnp.float32)


def _k4_body(z3p, g, d2, b2, out):
    t = (z3p[0, :, :DT] + z3p[0, :, DT:] + g[...]) * d2[...] + b2[...]
    m = jnp.max(t, axis=1, keepdims=True)
    lse = jnp.log(jnp.sum(jnp.exp(t - m), axis=1, keepdims=True)) + m
    out[...] = t - lse


def _rows(d):
    return pl.BlockSpec((RB, d), lambda i: (i, 0))


def _parts(ntab):
    return pl.BlockSpec((ntab, RB, 2 * DT), lambda i: (0, i, 0))


def _full(shape):
    return pl.BlockSpec(shape, lambda i: tuple(0 for _ in shape))


_k1 = pl.pallas_call(
    _k1_body,
    grid=(GRID,),
    in_specs=[pl.BlockSpec((RB, 2 * DW), lambda i: (i, 0)), _rows(DIN)],
    out_specs=[_rows(DT), _rows(DT), _rows(1), _rows(1)],
    out_shape=[
        jax.ShapeDtypeStruct((N, DT), jnp.float32),
        jax.ShapeDtypeStruct((N, DT), jnp.float32),
        jax.ShapeDtypeStruct((N, 1), jnp.float32),
        jax.ShapeDtypeStruct((N, 1), jnp.float32),
    ],
)

_k2 = pl.pallas_call(
    _k2_body,
    grid=(GRID,),
    in_specs=[_parts(2), _rows(1)],
    out_specs=[_rows(DT), _rows(DT)],
    out_shape=[
        jax.ShapeDtypeStruct((N, DT), jnp.float32),
        jax.ShapeDtypeStruct((N, DT), jnp.float32),
    ],
)

_k3 = pl.pallas_call(
    _k3_body,
    grid=(GRID,),
    in_specs=[_parts(2), _rows(1), _rows(1), _full((DIN, DH)),
              _full((1, DH)), _full((DH, DO))],
    out_specs=_rows(DO),
    out_shape=jax.ShapeDtypeStruct((N, DO), jnp.float32),
)

_k4 = pl.pallas_call(
    _k4_body,
    grid=(GRID,),
    in_specs=[_parts(1), _rows(DO), _rows(1), _full((1, DO))],
    out_specs=_rows(DO),
    out_shape=jax.ShapeDtypeStruct((N, DO), jnp.float32),
)


@jax.jit
def kernel(x, edge_index, W1, b1, W2, b2):
    src = edge_index[0]
    dst = edge_index[1]
    ar = np.arange(EPAD - E, dtype=np.int32)
    srcr = jnp.concatenate([src, jnp.asarray(ar % N)]).reshape(RPAD, LANE)
    dstr = jnp.concatenate([dst, jnp.asarray(N + (ar % GPAD))]).reshape(
        RPAD, LANE)

    onesc = jnp.ones((LANE, DW), jnp.float32)
    zcol = jnp.zeros((TROWS, DW), jnp.float32)
    zrow = jnp.zeros((LANE, DT), jnp.float32)

    prop1 = _make_prop(1)
    prop2 = _make_prop(2)

    degp = _make_deg()(dstr, onesc, zcol)
    y0lo, y0hi, d1, d2 = _k1(degp, x)
    z1p = prop2(y0lo, y0hi, srcr, dstr, zrow)
    y1lo, y1hi = _k2(z1p, d1)
    z2p = prop2(y1lo, y1hi, srcr, dstr, zrow)
    g = _k3(z2p, d1, d2, W1, b1.reshape(1, DH), W2)
    z3p = prop1(g, srcr, dstr, zrow)
    return _k4(z3p, g, d2, b2.reshape(1, DO))


# speedup vs baseline: 1.2470x; 1.2470x over previous
import math

import numpy as np

import jax
import jax.numpy as jnp
from jax import lax
from jax.experimental import pallas as pl
from jax.experimental.pallas import tpu as pltpu
from jax.experimental.pallas import tpu_sc as plsc

N = 10000
E = 320000
DIN = 128
DH = 128
DO = 64
DT = 64

LANE = 128
NW = 32
RW = 80
RPAD = NW * RW
EROWS = E // LANE
RREM = EROWS - (NW - 1) * RW
PROWS = RPAD - EROWS
GPAD = 112
NACC = N + GPAD
TROWS = NACC // 16
WCH = (128, 128, 128, 128, 120)

COEFF = math.exp(-1.0)
SELU_ALPHA = 1.6732632423543772848170429916717
SELU_SCALE = 1.0507009873554804934193349852946

_mesh = lambda: plsc.VectorSubcoreMesh(core_axis_name="c", subcore_axis_name="s")
_SC_PARAMS = pltpu.CompilerParams(use_tc_tiling_on_sc=False)


DW = 16


def _deg_body(dstr, dstp, onesc, zcol, out_hbm, didx, obuf, zbuf, acc):
    c = lax.axis_index("c")
    s = lax.axis_index("s")
    wid = s * 2 + c

    @pl.when(wid < NW - 1)
    def _():
        pltpu.sync_copy(dstr.at[pl.ds(wid * RW, RW)], didx)

    @pl.when(wid == NW - 1)
    def _():
        pltpu.sync_copy(dstr.at[pl.ds((NW - 1) * RW, RREM)],
                        didx.at[pl.ds(0, RREM)])
        pltpu.sync_copy(dstp, didx.at[pl.ds(RREM, PROWS)])

    pltpu.sync_copy(onesc, obuf)
    pltpu.sync_copy(zcol, zbuf)
    rbase = s * TROWS
    pltpu.sync_copy(zbuf, acc.at[pl.ds(rbase, TROWS)])
    plsc.subcore_barrier()

    def step(j, carry):
        pltpu.sync_copy(obuf, acc.at[didx.at[j]], add=True)
        return carry

    lax.fori_loop(0, RW, step, 0)
    plsc.subcore_barrier()
    pltpu.sync_copy(acc.at[pl.ds(rbase, TROWS)], zbuf)
    pltpu.sync_copy(zbuf, out_hbm.at[pl.ds(rbase, TROWS), pl.ds(DW * c, DW)])


def _make_deg():
    return pl.kernel(
        _deg_body,
        out_type=jax.ShapeDtypeStruct((NACC, 2 * DW), jnp.float32),
        mesh=_mesh(),
        compiler_params=_SC_PARAMS,
        scratch_types=[
            pltpu.VMEM((RW, LANE), jnp.int32),
            pltpu.VMEM((LANE, DW), jnp.float32),
            pltpu.VMEM((TROWS, DW), jnp.float32),
            pltpu.VMEM_SHARED((NACC, DW), jnp.float32),
        ],
    )


NBUF = 4


def _make_prop(ntab):
    def body(*refs):
        tabs = refs[:ntab]
        (srcr, srcp, dstr, dstp, zrow, out_hbm, sidx, didx,
         buf0, buf1, buf2, buf3,
         acc, sg0, sg1, sg2, sg3, ss0, ss1, ss2, ss3) = refs[ntab:]
        bufs = (buf0, buf1, buf2, buf3)
        sg = (sg0, sg1, sg2, sg3)
        ss = (ss0, ss1, ss2, ss3)
        c = lax.axis_index("c")
        s = lax.axis_index("s")
        wid = s * 2 + c
        base = wid * RW

        @pl.when(wid < NW - 1)
        def _():
            pltpu.sync_copy(srcr.at[pl.ds(base, RW)], sidx)
            pltpu.sync_copy(dstr.at[pl.ds(base, RW)], didx)

        @pl.when(wid == NW - 1)
        def _():
            pltpu.sync_copy(srcr.at[pl.ds((NW - 1) * RW, RREM)],
                            sidx.at[pl.ds(0, RREM)])
            pltpu.sync_copy(srcp, sidx.at[pl.ds(RREM, PROWS)])
            pltpu.sync_copy(dstr.at[pl.ds((NW - 1) * RW, RREM)],
                            didx.at[pl.ds(0, RREM)])
            pltpu.sync_copy(dstp, didx.at[pl.ds(RREM, PROWS)])
        rbase = s * TROWS
        pltpu.sync_copy(zrow, buf0)

        def zero_own_slice():
            off = 0
            for csz in WCH:
                pltpu.sync_copy(buf0.at[pl.ds(0, csz)],
                                acc.at[pl.ds(rbase + off, csz)])
                off += csz

        def gather(jj, b):
            return pltpu.async_copy(tab.at[sidx.at[jj]], bufs[b], sg[b])


        def scatter(jj, b):
            return pltpu.async_copy(bufs[b], acc.at[didx.at[jj]], ss[b],
                                    add=True)

        def wait_gather(jj, b):
            pltpu.make_async_copy(tab.at[sidx.at[jj]], bufs[b], sg[b]).wait()


        def wait_scatter(jj, b):
            pltpu.make_async_copy(bufs[b], acc.at[didx.at[jj]], ss[b]).wait()

        zero_own_slice()
        for t in range(ntab):
            tab = tabs[t]
            plsc.subcore_barrier()

            for b in range(2):
                gather(b, b)
            for b in range(4):
                if b >= 2:
                    wait_scatter(b - 2, (b + 2) % NBUF)
                if b + 2 < RW:
                    gather(b + 2, (b + 2) % NBUF)
                wait_gather(b, b)
                scatter(b, b)

            def step(i, carry):
                j0 = 4 * i
                for b in range(4):
                    j = j0 + b
                    wait_scatter(j - 2, (b + 2) % NBUF)

                    @pl.when(j + 2 < RW)
                    def _():
                        gather(j + 2, (b + 2) % NBUF)

                    wait_gather(j, b)
                    scatter(j, b)
                return carry

            lax.fori_loop(1, RW // 4, step, 0)
            wait_scatter(RW - 2, (RW - 2) % NBUF)
            wait_scatter(RW - 1, (RW - 1) % NBUF)
            plsc.subcore_barrier()
            off = 0
            for csz in WCH:
                pltpu.sync_copy(acc.at[pl.ds(rbase + off, csz)],
                                buf1.at[pl.ds(0, csz)])
                pltpu.sync_copy(
                    buf1.at[pl.ds(0, csz)],
                    out_hbm.at[t, pl.ds(rbase + off, csz), pl.ds(DT * c, DT)])
                off += csz
            if t + 1 < ntab:
                pltpu.sync_copy(zrow, buf0)
                zero_own_slice()

    return pl.kernel(
        body,
        out_type=jax.ShapeDtypeStruct((ntab, NACC, 2 * DT), jnp.float32),
        mesh=_mesh(),
        compiler_params=_SC_PARAMS,
        scratch_types=[
            pltpu.VMEM((RW, LANE), jnp.int32),
            pltpu.VMEM((RW, LANE), jnp.int32),
            pltpu.VMEM((LANE, DT), jnp.float32),
            pltpu.VMEM((LANE, DT), jnp.float32),
            pltpu.VMEM((LANE, DT), jnp.float32),
            pltpu.VMEM((LANE, DT), jnp.float32),
            pltpu.VMEM_SHARED((NACC, DT), jnp.float32),
            pltpu.SemaphoreType.DMA,
            pltpu.SemaphoreType.DMA,
            pltpu.SemaphoreType.DMA,
            pltpu.SemaphoreType.DMA,
            pltpu.SemaphoreType.DMA,
            pltpu.SemaphoreType.DMA,
            pltpu.SemaphoreType.DMA,
            pltpu.SemaphoreType.DMA,
        ],
    )


RB = 2000
GRID = N // RB


def _k1_body(degp, x, y0lo, y0hi, d1, d2):
    deg = degp[:, :1] + degp[:, DW:DW + 1]
    dinv1 = jnp.where(deg > 0, lax.rsqrt(jnp.maximum(deg, 1e-12)), 0.0)
    d1[...] = dinv1
    d2[...] = lax.rsqrt(deg + 1.0)
    y0 = x[...] * dinv1
    y0lo[...] = y0[:, :DT]
    y0hi[...] = y0[:, DT:]


def _k2_body(z1p, d1, y1lo, y1hi):
    dd = d1[...] * d1[...]
    y1lo[...] = (z1p[0, :, :DT] + z1p[0, :, DT:]) * dd
    y1hi[...] = (z1p[1, :, :DT] + z1p[1, :, DT:]) * dd


def _k3_body(z2p, d1, d2, W1, b1, W2, g):
    s = d1[...] * COEFF
    x2 = jnp.concatenate(
        [z2p[0, :, :DT] + z2p[0, :, DT:],
         z2p[1, :, :DT] + z2p[1, :, DT:]], axis=1) * s
    h = jnp.dot(x2, W1[...], preferred_element_type=jnp.float32) + b1[...]
    h = SELU_SCALE * jnp.where(h > 0, h, SELU_ALPHA * (jnp.exp(h) - 1.0))
    g[...] = d2[...] * jnp.dot(h, W2[...], preferred_element_type=jnp.float32)


def _k4_body(z3p, g, d2, b2, out):
    t = (z3p[0, :, :DT] + z3p[0, :, DT:] + g[...]) * d2[...] + b2[...]
    m = jnp.max(t, axis=1, keepdims=True)
    lse = jnp.log(jnp.sum(jnp.exp(t - m), axis=1, keepdims=True)) + m
    out[...] = t - lse


def _rows(d):
    return pl.BlockSpec((RB, d), lambda i: (i, 0))


def _parts(ntab):
    return pl.BlockSpec((ntab, RB, 2 * DT), lambda i: (0, i, 0))


def _full(shape):
    return pl.BlockSpec(shape, lambda i: tuple(0 for _ in shape))


_k1 = pl.pallas_call(
    _k1_body,
    grid=(GRID,),
    in_specs=[pl.BlockSpec((RB, 2 * DW), lambda i: (i, 0)), _rows(DIN)],
    out_specs=[_rows(DT), _rows(DT), _rows(1), _rows(1)],
    out_shape=[
        jax.ShapeDtypeStruct((N, DT), jnp.float32),
        jax.ShapeDtypeStruct((N, DT), jnp.float32),
        jax.ShapeDtypeStruct((N, 1), jnp.float32),
        jax.ShapeDtypeStruct((N, 1), jnp.float32),
    ],
)

_k2 = pl.pallas_call(
    _k2_body,
    grid=(GRID,),
    in_specs=[_parts(2), _rows(1)],
    out_specs=[_rows(DT), _rows(DT)],
    out_shape=[
        jax.ShapeDtypeStruct((N, DT), jnp.float32),
        jax.ShapeDtypeStruct((N, DT), jnp.float32),
    ],
)

_k3 = pl.pallas_call(
    _k3_body,
    grid=(GRID,),
    in_specs=[_parts(2), _rows(1), _rows(1), _full((DIN, DH)),
              _full((1, DH)), _full((DH, DO))],
    out_specs=_rows(DO),
    out_shape=jax.ShapeDtypeStruct((N, DO), jnp.float32),
)

_k4 = pl.pallas_call(
    _k4_body,
    grid=(GRID,),
    in_specs=[_parts(1), _rows(DO), _rows(1), _full((1, DO))],
    out_specs=_rows(DO),
    out_shape=jax.ShapeDtypeStruct((N, DO), jnp.float32),
)


@jax.jit
def kernel(x, edge_index, W1, b1, W2, b2):
    srcr = edge_index[0].reshape(EROWS, LANE)
    dstr = edge_index[1].reshape(EROWS, LANE)
    ar = np.arange(PROWS * LANE, dtype=np.int32)
    srcp = jnp.asarray((ar % N).reshape(PROWS, LANE))
    dstp = jnp.asarray((N + ar % GPAD).reshape(PROWS, LANE))

    onesc = jnp.ones((LANE, DW), jnp.float32)
    zcol = jnp.zeros((TROWS, DW), jnp.float32)
    zrow = jnp.zeros((LANE, DT), jnp.float32)

    prop1 = _make_prop(1)
    prop2 = _make_prop(2)

    degp = _make_deg()(dstr, dstp, onesc, zcol)
    y0lo, y0hi, d1, d2 = _k1(degp, x)
    z1p = prop2(y0lo, y0hi, srcr, srcp, dstr, dstp, zrow)
    y1lo, y1hi = _k2(z1p, d1)
    z2p = prop2(y1lo, y1hi, srcr, srcp, dstr, dstp, zrow)
    g = _k3(z2p, d1, d2, W1, b1.reshape(1, DH), W2)
    z3p = prop1(g, srcr, srcp, dstr, dstp, zrow)
    return _k4(z3p, g, d2, b2.reshape(1, DO))


# speedup vs baseline: 1.2740x; 1.0217x over previous
import math

import numpy as np

import jax
import jax.numpy as jnp
from jax import lax
from jax.experimental import pallas as pl
from jax.experimental.pallas import tpu as pltpu
from jax.experimental.pallas import tpu_sc as plsc

N = 10000
E = 320000
DIN = 128
DH = 128
DO = 64
DT = 64

LANE = 128
NW = 32
RW = 80
RPAD = NW * RW
EPAD = RPAD * LANE
GPAD = 112
NACC = N + GPAD
TROWS = NACC // 16
WCH = (128, 128, 128, 128, 120)

COEFF = math.exp(-1.0)
SELU_ALPHA = 1.6732632423543772848170429916717
SELU_SCALE = 1.0507009873554804934193349852946

_mesh = lambda: plsc.VectorSubcoreMesh(core_axis_name="c", subcore_axis_name="s")
_SC_PARAMS = pltpu.CompilerParams(use_tc_tiling_on_sc=False)


DW = 16


def _deg_body(dstr, onesc, zcol, out_hbm, didx, obuf, zbuf, acc):
    c = lax.axis_index("c")
    s = lax.axis_index("s")
    wid = s * 2 + c
    pltpu.sync_copy(dstr.at[pl.ds(wid * RW, RW)], didx)
    pltpu.sync_copy(onesc, obuf)
    pltpu.sync_copy(zcol, zbuf)
    rbase = s * TROWS
    pltpu.sync_copy(zbuf, acc.at[pl.ds(rbase, TROWS)])
    plsc.subcore_barrier()

    def step(j, carry):
        pltpu.sync_copy(obuf, acc.at[didx.at[j]], add=True)
        return carry

    lax.fori_loop(0, RW, step, 0)
    plsc.subcore_barrier()
    pltpu.sync_copy(acc.at[pl.ds(rbase, TROWS)], zbuf)
    pltpu.sync_copy(zbuf, out_hbm.at[pl.ds(rbase, TROWS), pl.ds(DW * c, DW)])


def _make_deg():
    return pl.kernel(
        _deg_body,
        out_type=jax.ShapeDtypeStruct((NACC, 2 * DW), jnp.float32),
        mesh=_mesh(),
        compiler_params=_SC_PARAMS,
        scratch_types=[
            pltpu.VMEM((RW, LANE), jnp.int32),
            pltpu.VMEM((LANE, DW), jnp.float32),
            pltpu.VMEM((TROWS, DW), jnp.float32),
            pltpu.VMEM_SHARED((NACC, DW), jnp.float32),
        ],
    )


NBUF = 4


def _make_prop(ntab):
    def body(*refs):
        tabs = refs[:ntab]
        (srcr, dstr, zrow, out_hbm, sidx, didx, buf0, buf1, buf2, buf3,
         acc, sg0, sg1, sg2, sg3, ss0, ss1, ss2, ss3) = refs[ntab:]
        bufs = (buf0, buf1, buf2, buf3)
        sg = (sg0, sg1, sg2, sg3)
        ss = (ss0, ss1, ss2, ss3)
        c = lax.axis_index("c")
        s = lax.axis_index("s")
        wid = s * 2 + c
        base = wid * RW
        pltpu.sync_copy(srcr.at[pl.ds(base, RW)], sidx)
        pltpu.sync_copy(dstr.at[pl.ds(base, RW)], didx)
        rbase = s * TROWS
        pltpu.sync_copy(zrow, buf0)

        def zero_own_slice():
            off = 0
            for csz in WCH:
                pltpu.sync_copy(buf0.at[pl.ds(0, csz)],
                                acc.at[pl.ds(rbase + off, csz)])
                off += csz

        def gather(jj, b):
            return pltpu.async_copy(tab.at[sidx.at[jj]], bufs[b], sg[b])


        def scatter(jj, b):
            return pltpu.async_copy(bufs[b], acc.at[didx.at[jj]], ss[b],
                                    add=True)

        def wait_gather(jj, b):
            pltpu.make_async_copy(tab.at[sidx.at[jj]], bufs[b], sg[b]).wait()


        def wait_scatter(jj, b):
            pltpu.make_async_copy(bufs[b], acc.at[didx.at[jj]], ss[b]).wait()

        zero_own_slice()
        for t in range(ntab):
            tab = tabs[t]
            plsc.subcore_barrier()

            for b in range(2):
                gather(b, b)
            for b in range(4):
                if b >= 2:
                    wait_scatter(b - 2, (b + 2) % NBUF)
                if b + 2 < RW:
                    gather(b + 2, (b + 2) % NBUF)
                wait_gather(b, b)
                scatter(b, b)

            def step(i, carry):
                j0 = 4 * i
                for b in range(4):
                    j = j0 + b
                    wait_scatter(j - 2, (b + 2) % NBUF)

                    @pl.when(j + 2 < RW)
                    def _():
                        gather(j + 2, (b + 2) % NBUF)

                    wait_gather(j, b)
                    scatter(j, b)
                return carry

            lax.fori_loop(1, RW // 4, step, 0)
            wait_scatter(RW - 2, (RW - 2) % NBUF)
            wait_scatter(RW - 1, (RW - 1) % NBUF)
            plsc.subcore_barrier()
            off = 0
            for csz in WCH:
                pltpu.sync_copy(acc.at[pl.ds(rbase + off, csz)],
                                buf1.at[pl.ds(0, csz)])
                pltpu.sync_copy(
                    buf1.at[pl.ds(0, csz)],
                    out_hbm.at[t, pl.ds(rbase + off, csz), pl.ds(DT * c, DT)])
                off += csz
            if t + 1 < ntab:
                pltpu.sync_copy(zrow, buf0)
                zero_own_slice()

    return pl.kernel(
        body,
        out_type=jax.ShapeDtypeStruct((ntab, NACC, 2 * DT), jnp.float32),
        mesh=_mesh(),
        compiler_params=_SC_PARAMS,
        scratch_types=[
            pltpu.VMEM((RW, LANE), jnp.int32),
            pltpu.VMEM((RW, LANE), jnp.int32),
            pltpu.VMEM((LANE, DT), jnp.float32),
            pltpu.VMEM((LANE, DT), jnp.float32),
            pltpu.VMEM((LANE, DT), jnp.float32),
            pltpu.VMEM((LANE, DT), jnp.float32),
            pltpu.VMEM_SHARED((NACC, DT), jnp.float32),
            pltpu.SemaphoreType.DMA,
            pltpu.SemaphoreType.DMA,
            pltpu.SemaphoreType.DMA,
            pltpu.SemaphoreType.DMA,
            pltpu.SemaphoreType.DMA,
            pltpu.SemaphoreType.DMA,
            pltpu.SemaphoreType.DMA,
            pltpu.SemaphoreType.DMA,
        ],
    )


RB = 2000
GRID = N // RB


def _k1_body(degp, x, y0lo, y0hi, d1, d2):
    deg = degp[:, :1] + degp[:, DW:DW + 1]
    dinv1 = jnp.where(deg > 0, lax.rsqrt(jnp.maximum(deg, 1e-12)), 0.0)
    d1[...] = dinv1
    d2[...] = lax.rsqrt(deg + 1.0)
    y0 = x[...] * dinv1
    y0lo[...] = y0[:, :DT]
    y0hi[...] = y0[:, DT:]


def _k2_body(z1p, d1, y1lo, y1hi):
    dd = d1[...] * d1[...]
    y1lo[...] = (z1p[0, :, :DT] + z1p[0, :, DT:]) * dd
    y1hi[...] = (z1p[1, :, :DT] + z1p[1, :, DT:]) * dd


def _k3_body(z2p, d1, d2, W1, b1, W2, g):
    s = d1[...] * COEFF
    x2 = jnp.concatenate(
        [z2p[0, :, :DT] + z2p[0, :, DT:],
         z2p[1, :, :DT] + z2p[1, :, DT:]], axis=1) * s
    h = jnp.dot(x2, W1[...], preferred_element_type=jnp.float32) + b1[...]
    h = SELU_SCALE * jnp.where(h > 0, h, SELU_ALPHA * (jnp.exp(h) - 1.0))
    g[...] = d2[...] * jnp.dot(h, W2[...], preferred_element_type=jnp.float32)


def _k4_body(z3p, g, d2, b2, out):
    t = (z3p[0, :, :DT] + z3p[0, :, DT:] + g[...]) * d2[...] + b2[...]
    m = jnp.max(t, axis=1, keepdims=True)
    lse = jnp.log(jnp.sum(jnp.exp(t - m), axis=1, keepdims=True)) + m
    out[...] = t - lse


def _rows(d):
    return pl.BlockSpec((RB, d), lambda i: (i, 0))


def _parts(ntab):
    return pl.BlockSpec((ntab, RB, 2 * DT), lambda i: (0, i, 0))


def _full(shape):
    return pl.BlockSpec(shape, lambda i: tuple(0 for _ in shape))


_k1 = pl.pallas_call(
    _k1_body,
    grid=(GRID,),
    in_specs=[pl.BlockSpec((RB, 2 * DW), lambda i: (i, 0)), _rows(DIN)],
    out_specs=[_rows(DT), _rows(DT), _rows(1), _rows(1)],
    out_shape=[
        jax.ShapeDtypeStruct((N, DT), jnp.float32),
        jax.ShapeDtypeStruct((N, DT), jnp.float32),
        jax.ShapeDtypeStruct((N, 1), jnp.float32),
        jax.ShapeDtypeStruct((N, 1), jnp.float32),
    ],
)

_k2 = pl.pallas_call(
    _k2_body,
    grid=(GRID,),
    in_specs=[_parts(2), _rows(1)],
    out_specs=[_rows(DT), _rows(DT)],
    out_shape=[
        jax.ShapeDtypeStruct((N, DT), jnp.float32),
        jax.ShapeDtypeStruct((N, DT), jnp.float32),
    ],
)

_k3 = pl.pallas_call(
    _k3_body,
    grid=(GRID,),
    in_specs=[_parts(2), _rows(1), _rows(1), _full((DIN, DH)),
              _full((1, DH)), _full((DH, DO))],
    out_specs=_rows(DO),
    out_shape=jax.ShapeDtypeStruct((N, DO), jnp.float32),
)

_k4 = pl.pallas_call(
    _k4_body,
    grid=(GRID,),
    in_specs=[_parts(1), _rows(DO), _rows(1), _full((1, DO))],
    out_specs=_rows(DO),
    out_shape=jax.ShapeDtypeStruct((N, DO), jnp.float32),
)


@jax.jit
def kernel(x, edge_index, W1, b1, W2, b2):
    src = edge_index[0]
    dst = edge_index[1]
    ar = np.arange(EPAD - E, dtype=np.int32)
    srcr = jnp.concatenate([src, jnp.asarray(ar % N)]).reshape(RPAD, LANE)
    dstr = jnp.concatenate([dst, jnp.asarray(N + (ar % GPAD))]).reshape(
        RPAD, LANE)

    onesc = jnp.ones((LANE, DW), jnp.float32)
    zcol = jnp.zeros((TROWS, DW), jnp.float32)
    zrow = jnp.zeros((LANE, DT), jnp.float32)

    prop1 = _make_prop(1)
    prop2 = _make_prop(2)

    degp = _make_deg()(dstr, onesc, zcol)
    y0lo, y0hi, d1, d2 = _k1(degp, x)
    z1p = prop2(y0lo, y0hi, srcr, dstr, zrow)
    y1lo, y1hi = _k2(z1p, d1)
    z2p = prop2(y1lo, y1hi, srcr, dstr, zrow)
    g = _k3(z2p, d1, d2, W1, b1.reshape(1, DH), W2)
    z3p = prop1(g, srcr, dstr, zrow)
    return _k4(z3p, g, d2, b2.reshape(1, DO))
